# baseline (speedup 1.0000x reference)
import functools

import jax
import jax.numpy as jnp
from jax import lax
from jax.experimental import pallas as pl
from jax.experimental.pallas import tpu as pltpu
from jax.experimental.pallas import tpu_sc as plsc

_NC = 2
_NS = 16
_NW = _NC * _NS
_LANES = 16


@functools.lru_cache(maxsize=None)
def _make_pool(B, CTX, D):
    assert B % _NW == 0 and D % _LANES == 0
    bpw = B // _NW
    nd = D // _LANES
    scale = 1.0 / CTX
    mesh = plsc.VectorSubcoreMesh(
        core_axis_name="c", subcore_axis_name="s",
        num_cores=_NC, num_subcores=_NS)

    @functools.partial(
        pl.kernel,
        out_type=jax.ShapeDtypeStruct((B, D), jnp.float32),
        mesh=mesh,
        scratch_types=[
            pltpu.VMEM((bpw, CTX), jnp.int32),
            pltpu.VMEM((CTX, D), jnp.float32),
            pltpu.VMEM((bpw, D), jnp.float32),
            pltpu.SemaphoreType.DMA,
        ],
        compiler_params=pltpu.CompilerParams(use_tc_tiling_on_sc=False),
    )
    def pool(ctx_hbm, emb_hbm, out_hbm, idx_v, rows_v, pooled_v, sem):
        wid = lax.axis_index("s") * _NC + lax.axis_index("c")
        base = wid * bpw
        pltpu.sync_copy(ctx_hbm.at[pl.ds(base, bpw)], idx_v)

        def body(i, carry):
            pltpu.async_copy(emb_hbm.at[idx_v.at[i]], rows_v, sem).wait()

            def rbody(j, accs):
                return tuple(
                    accs[d] + rows_v[j, pl.ds(_LANES * d, _LANES)]
                    for d in range(nd))

            zero = jnp.zeros((_LANES,), jnp.float32)
            accs = lax.fori_loop(0, CTX, rbody, (zero,) * nd)
            for d in range(nd):
                pooled_v[i, pl.ds(_LANES * d, _LANES)] = accs[d] * scale
            return carry

        lax.fori_loop(0, bpw, body, 0)
        pltpu.sync_copy(pooled_v, out_hbm.at[pl.ds(base, bpw)])

    return pool


@functools.lru_cache(maxsize=None)
def _make_proj(B, D, V, bb, bv):
    def proj_body(x_ref, w_ref, b_ref, o_ref):
        o_ref[...] = lax.dot_general(
            x_ref[...], w_ref[...],
            (((1,), (1,)), ((), ())),
            preferred_element_type=jnp.float32,
        ) + b_ref[...]

    return pl.pallas_call(
        proj_body,
        grid=(pl.cdiv(V, bv), pl.cdiv(B, bb)),
        in_specs=[
            pl.BlockSpec((bb, D), lambda i, j: (j, 0)),
            pl.BlockSpec((bv, D), lambda i, j: (i, 0)),
            pl.BlockSpec((1, bv), lambda i, j: (0, i)),
        ],
        out_specs=pl.BlockSpec((bb, bv), lambda i, j: (j, i)),
        out_shape=jax.ShapeDtypeStruct((B, V), jnp.float32),
    )


def kernel(context, emb_table, lin_w, lin_b):
    B, CTX = context.shape
    V, D = emb_table.shape
    pooled = _make_pool(B, CTX, D)(context.astype(jnp.int32), emb_table)
    return _make_proj(B, D, V, 256, 2048)(pooled, lin_w, lin_b.reshape(1, V))


# speedup vs baseline: 1.1532x; 1.1532x over previous
import functools

import jax
import jax.numpy as jnp
from jax import lax
from jax.experimental import pallas as pl
from jax.experimental.pallas import tpu as pltpu
from jax.experimental.pallas import tpu_sc as plsc

_NC = 2
_NS = 16
_NW = _NC * _NS
_LANES = 16


@functools.lru_cache(maxsize=None)
def _make_pool(B, CTX, D):
    assert B % _NW == 0 and D % _LANES == 0
    bpw = B // _NW
    nd = D // _LANES
    scale = 1.0 / CTX
    mesh = plsc.VectorSubcoreMesh(
        core_axis_name="c", subcore_axis_name="s",
        num_cores=_NC, num_subcores=_NS)

    @functools.partial(
        pl.kernel,
        out_type=jax.ShapeDtypeStruct((B, D), jnp.float32),
        mesh=mesh,
        scratch_types=[
            pltpu.VMEM((bpw, CTX), jnp.int32),
            pltpu.VMEM((CTX, D), jnp.float32),
            pltpu.VMEM((bpw, D), jnp.float32),
            pltpu.SemaphoreType.DMA,
        ],
        compiler_params=pltpu.CompilerParams(use_tc_tiling_on_sc=False),
    )
    def pool(ctx_hbm, emb_hbm, out_hbm, idx_v, rows_v, pooled_v, sem):
        wid = lax.axis_index("s") * _NC + lax.axis_index("c")
        base = wid * bpw
        pltpu.sync_copy(ctx_hbm.at[pl.ds(base, bpw)], idx_v)

        def body(i, carry):
            pltpu.async_copy(emb_hbm.at[idx_v.at[i]], rows_v, sem).wait()

            def rbody(j, accs):
                return tuple(
                    accs[d] + rows_v[j, pl.ds(_LANES * d, _LANES)]
                    for d in range(nd))

            zero = jnp.zeros((_LANES,), jnp.float32)
            accs = lax.fori_loop(0, CTX, rbody, (zero,) * nd)
            for d in range(nd):
                pooled_v[i, pl.ds(_LANES * d, _LANES)] = accs[d] * scale
            return carry

        lax.fori_loop(0, bpw, body, 0)
        pltpu.sync_copy(pooled_v, out_hbm.at[pl.ds(base, bpw)])

    return pool


@functools.lru_cache(maxsize=None)
def _make_proj(B, D, V, bv):
    def proj_body(x_ref, w_ref, b_ref, o_ref):
        o_ref[...] = lax.dot_general(
            x_ref[...], w_ref[...],
            (((1,), (1,)), ((), ())),
            preferred_element_type=jnp.float32,
        ) + b_ref[...]

    return pl.pallas_call(
        proj_body,
        grid=(pl.cdiv(V, bv),),
        in_specs=[
            pl.BlockSpec((B, D), lambda i: (0, 0)),
            pl.BlockSpec((bv, D), lambda i: (i, 0)),
            pl.BlockSpec((1, bv), lambda i: (0, i)),
        ],
        out_specs=pl.BlockSpec((B, bv), lambda i: (0, i)),
        out_shape=jax.ShapeDtypeStruct((B, V), jnp.float32),
    )


def kernel(context, emb_table, lin_w, lin_b):
    B, CTX = context.shape
    V, D = emb_table.shape
    pooled = _make_pool(B, CTX, D)(context.astype(jnp.int32), emb_table)
    return _make_proj(B, D, V, 4096)(pooled, lin_w, lin_b.reshape(1, V))


# speedup vs baseline: 1.1624x; 1.0080x over previous
import functools

import jax
import jax.numpy as jnp
from jax import lax
from jax.experimental import pallas as pl
from jax.experimental.pallas import tpu as pltpu
from jax.experimental.pallas import tpu_sc as plsc

_NC = 2
_NS = 16
_NW = _NC * _NS
_LANES = 16


@functools.lru_cache(maxsize=None)
def _make_pool(B, CTX, D):
    assert B % _NW == 0 and D % _LANES == 0
    bpw = B // _NW
    nd = D // _LANES
    scale = 1.0 / CTX
    mesh = plsc.VectorSubcoreMesh(
        core_axis_name="c", subcore_axis_name="s",
        num_cores=_NC, num_subcores=_NS)

    @functools.partial(
        pl.kernel,
        out_type=jax.ShapeDtypeStruct((B, D), jnp.float32),
        mesh=mesh,
        scratch_types=[
            pltpu.VMEM((bpw, CTX), jnp.int32),
            pltpu.VMEM((CTX, D), jnp.float32),
            pltpu.VMEM((CTX, D), jnp.float32),
            pltpu.VMEM((bpw, D), jnp.float32),
            pltpu.SemaphoreType.DMA,
            pltpu.SemaphoreType.DMA,
        ],
        compiler_params=pltpu.CompilerParams(use_tc_tiling_on_sc=False),
    )
    def pool(ctx_hbm, emb_hbm, out_hbm, idx_v, rows0, rows1, pooled_v,
             sem0, sem1):
        wid = lax.axis_index("s") * _NC + lax.axis_index("c")
        base = wid * bpw
        pltpu.sync_copy(ctx_hbm.at[pl.ds(base, bpw)], idx_v)

        def reduce_into(rows, i):
            def rbody(j, accs):
                return tuple(
                    accs[d] + rows[j, pl.ds(_LANES * d, _LANES)]
                    for d in range(nd))

            zero = jnp.zeros((_LANES,), jnp.float32)
            accs = lax.fori_loop(0, CTX, rbody, (zero,) * nd)
            for d in range(nd):
                pooled_v[i, pl.ds(_LANES * d, _LANES)] = accs[d] * scale

        pltpu.async_copy(emb_hbm.at[idx_v.at[0]], rows0, sem0)

        def body(p, carry):
            i0 = 2 * p
            pltpu.make_async_copy(
                emb_hbm.at[idx_v.at[i0]], rows0, sem0).wait()
            pltpu.async_copy(emb_hbm.at[idx_v.at[i0 + 1]], rows1, sem1)
            reduce_into(rows0, i0)
            pltpu.make_async_copy(
                emb_hbm.at[idx_v.at[i0 + 1]], rows1, sem1).wait()

            @pl.when(p < bpw // 2 - 1)
            def _():
                pltpu.async_copy(
                    emb_hbm.at[idx_v.at[i0 + 2]], rows0, sem0)
            reduce_into(rows1, i0 + 1)
            return carry

        lax.fori_loop(0, bpw // 2, body, 0)
        pltpu.sync_copy(pooled_v, out_hbm.at[pl.ds(base, bpw)])

    return pool


@functools.lru_cache(maxsize=None)
def _make_proj(B, D, V, bv):
    def proj_body(x_ref, w_ref, b_ref, o_ref):
        o_ref[...] = lax.dot_general(
            x_ref[...], w_ref[...],
            (((1,), (1,)), ((), ())),
            preferred_element_type=jnp.float32,
        ) + b_ref[...]

    return pl.pallas_call(
        proj_body,
        grid=(pl.cdiv(V, bv),),
        in_specs=[
            pl.BlockSpec((B, D), lambda i: (0, 0)),
            pl.BlockSpec((bv, D), lambda i: (i, 0)),
            pl.BlockSpec((1, bv), lambda i: (0, i)),
        ],
        out_specs=pl.BlockSpec((B, bv), lambda i: (0, i)),
        out_shape=jax.ShapeDtypeStruct((B, V), jnp.float32),
    )


def kernel(context, emb_table, lin_w, lin_b):
    B, CTX = context.shape
    V, D = emb_table.shape
    pooled = _make_pool(B, CTX, D)(context.astype(jnp.int32), emb_table)
    return _make_proj(B, D, V, 4096)(pooled, lin_w, lin_b.reshape(1, V))
